# baseline (speedup 1.0000x reference)
import functools

import jax
import jax.numpy as jnp
from jax import lax
from jax.experimental import pallas as pl
from jax.experimental.pallas import tpu as pltpu
from jax.experimental.pallas import tpu_sc as plsc

P = 128
D = 24576
B = 16 * 128
NC, NS = 2, 16
HALF = D // NC
RPT = B // NS
RLD = P // NS
RPC = 1
NCH = RPT // RPC

_mesh = plsc.VectorSubcoreMesh(core_axis_name="c", subcore_axis_name="s")


@functools.partial(
    pl.kernel,
    mesh=_mesh,
    out_type=jax.ShapeDtypeStruct((B, NC, HALF), jnp.float32),
    scratch_types=[
        pltpu.VMEM_SHARED((P, HALF), jnp.float32),
        pltpu.VMEM((RPT,), jnp.int32),
        pltpu.VMEM((RPC, HALF), jnp.float32),
        pltpu.SemaphoreType.DMA,
    ],
)
def _gather(idx_hbm, table_hbm, out_hbm, cache, idx_v, buf0, sem):
    c = lax.axis_index("c")
    s = lax.axis_index("s")
    pltpu.sync_copy(
        table_hbm.at[pl.ds(s * RLD, RLD), pl.ds(c * HALF, HALF)],
        cache.at[pl.ds(s * RLD, RLD)],
    )
    pltpu.sync_copy(idx_hbm.at[s], idx_v)
    plsc.subcore_barrier()

    base = s * RPT

    def body(g, carry):
        vec = idx_v[pl.ds(g * 16, 16)]
        for k in range(16):
            i = g * 16 + k
            pltpu.async_copy(cache.at[pl.ds(vec[k], 1)], buf0, sem).wait()
            pltpu.sync_copy(buf0, out_hbm.at[pl.ds(base + i, 1), c])
        return carry

    lax.fori_loop(0, RPT // 16, body, 0)


def kernel(prefix, table):
    idx = prefix.reshape(NS, RPT).astype(jnp.int32)
    out = _gather(idx, table)
    return out.reshape(prefix.shape[0], prefix.shape[1], D)


# speedup vs baseline: 4.8117x; 4.8117x over previous
import functools

import jax
import jax.numpy as jnp
from jax import lax
from jax.experimental import pallas as pl
from jax.experimental.pallas import tpu as pltpu
from jax.experimental.pallas import tpu_sc as plsc

P = 128
D = 24576
B = 16 * 128
NC, NS = 2, 16
NW = NC * NS
OWN = P // NW

_mesh = plsc.VectorSubcoreMesh(core_axis_name="c", subcore_axis_name="s")


@functools.partial(
    pl.kernel,
    mesh=_mesh,
    out_type=jax.ShapeDtypeStruct((B, D), jnp.float32),
    scratch_types=[
        pltpu.VMEM((OWN, D), jnp.float32),
        pltpu.VMEM((B,), jnp.int32),
        pltpu.SemaphoreType.DMA,
    ],
)
def _gather(idx_hbm, table_hbm, out_hbm, cache, idx_v, sem):
    w = lax.axis_index("s") * NC + lax.axis_index("c")
    lo = w * OWN
    pltpu.sync_copy(table_hbm.at[pl.ds(lo, OWN)], cache)
    pltpu.sync_copy(idx_hbm, idx_v)

    def body(g, cnt):
        vec = idx_v[pl.ds(g * 16, 16)]
        for k in range(16):
            rel = vec[k] - lo
            mine = (rel >= 0) & (rel < OWN)

            @pl.when(mine)
            def _():
                pltpu.async_copy(
                    cache.at[pl.ds(rel, 1)],
                    out_hbm.at[pl.ds(g * 16 + k, 1)],
                    sem,
                )

            cnt = cnt + mine.astype(jnp.int32)
        return cnt

    cnt = lax.fori_loop(0, B // 16, body, jnp.int32(0))

    def drain(i, carry):
        pltpu.make_async_copy(
            cache.at[pl.ds(0, 1)],
            out_hbm.at[pl.ds(0, 1)],
            sem,
        ).wait()
        return carry

    lax.fori_loop(0, cnt, drain, 0)


def kernel(prefix, table):
    idx = prefix.reshape(B).astype(jnp.int32)
    out = _gather(idx, table)
    return out.reshape(prefix.shape[0], prefix.shape[1], D)
